# Initial kernel scaffold; baseline (speedup 1.0000x reference)
import jax
import jax.numpy as jnp
from jax.experimental import pallas as pl


def kernel(x1, x2, embed, W1, b1, W2, b2):
    raise NotImplementedError("write your pallas kernel here")


# speedup vs baseline: 6.7161x; 6.7161x over previous
import functools

import jax
import jax.numpy as jnp
from jax.experimental import pallas as pl
from jax.experimental.pallas import tpu as pltpu

P = 53
D_EMBED = 128
HIDDEN = 256
B = 16384
BLK = 2048


def _fused_body(x1_ref, x2_ref, embed_ref, W1_ref, b1_ref, W2_ref, b2_ref,
                out_ref, m12_ref):
    i = pl.program_id(0)

    @pl.when(i == 0)
    def _prep():
        e = embed_ref[...]
        m1 = jnp.dot(e, W1_ref[0:D_EMBED, :],
                     preferred_element_type=jnp.float32)
        m2 = jnp.dot(e, W1_ref[D_EMBED:2 * D_EMBED, :],
                     preferred_element_type=jnp.float32)
        z = jnp.zeros((64 - P, HIDDEN), dtype=jnp.float32)
        m12_ref[...] = jnp.concatenate([m1, z, m2, z], axis=0)

    xb1 = x1_ref[0, 0, :]
    xb2 = x2_ref[0, 0, :]
    cols = jax.lax.broadcasted_iota(jnp.int32, (BLK, 2 * 64), 1)
    onehot = ((cols == xb1[:, None]) | (cols == (xb2[:, None] + 64))
              ).astype(jnp.float32)
    g = jnp.dot(onehot, m12_ref[...],
                preferred_element_type=jnp.float32)
    h = jnp.maximum(g + b1_ref[0, :], 0.0)
    out_ref[...] = jnp.dot(h, W2_ref[...],
                           preferred_element_type=jnp.float32) + b2_ref[0, :]


@jax.jit
def kernel(x1, x2, embed, W1, b1, W2, b2):
    nb = B // BLK
    x1r = x1.reshape(nb, 1, BLK)
    x2r = x2.reshape(nb, 1, BLK)
    return pl.pallas_call(
        _fused_body,
        grid=(nb,),
        in_specs=[
            pl.BlockSpec((1, 1, BLK), lambda i: (i, 0, 0)),
            pl.BlockSpec((1, 1, BLK), lambda i: (i, 0, 0)),
            pl.BlockSpec((P, D_EMBED), lambda i: (0, 0)),
            pl.BlockSpec((2 * D_EMBED, HIDDEN), lambda i: (0, 0)),
            pl.BlockSpec((1, HIDDEN), lambda i: (0, 0)),
            pl.BlockSpec((HIDDEN, P), lambda i: (0, 0)),
            pl.BlockSpec((1, P), lambda i: (0, 0)),
        ],
        out_specs=pl.BlockSpec((BLK, P), lambda i: (i, 0)),
        out_shape=jax.ShapeDtypeStruct((B, P), jnp.float32),
        scratch_shapes=[pltpu.VMEM((2 * 64, HIDDEN), jnp.float32)],
    )(x1r, x2r, embed, W1, b1.reshape(1, HIDDEN), W2, b2.reshape(1, P))
